# baseline (speedup 1.0000x reference)
import functools

import jax
import jax.numpy as jnp
from jax import lax
from jax.experimental import pallas as pl
from jax.experimental.pallas import tpu as pltpu
from jax.experimental.pallas import tpu_sc as plsc

NC = 2
NS = 16
L = 16
NW = NC * NS

NUM_IMGS = 16
SPATIAL = 512 * 512
COLS_PER_TILE = SPATIAL // NW
CH = 4096
HALVES = COLS_PER_TILE // CH
STEPS = NUM_IMGS * HALVES
VREGS = CH // L


def _sc_body(pred_ref, target_ref, input_ref, out_ref, buf0, buf1, acc,
             sem0, sem1):
    wid = lax.axis_index("s") * NC + lax.axis_index("c")
    col_base = pl.multiple_of(wid * COLS_PER_TILE, 8)
    arrs = (pred_ref, target_ref, input_ref)
    bufs = (buf0, buf1)
    sems = (sem0, sem1)

    zero = jnp.zeros((L,), jnp.float32)
    for j in range(768 // L):
        acc[pl.ds(j * L, L)] = zero

    lane = lax.iota(jnp.int32, L)
    ones = jnp.ones((L,), jnp.float32)

    def start_dmas(step):
        n, h = divmod(step, HALVES)
        slot = step % 2
        col0 = pl.multiple_of(col_base + h * CH, 8)
        cps = []
        for k in range(3):
            for c in range(3):
                cp = pltpu.make_async_copy(
                    arrs[k].at[n * 3 + c, pl.ds(col0, CH)],
                    bufs[slot].at[pl.ds((k * 3 + c) * CH, CH)],
                    sems[slot])
                cp.start()
                cps.append(cp)
        return cps

    def compute(step):
        buf = bufs[step % 2]

        def px(i, _):
            off = i * L

            def luma(k):
                return (0.299 * buf[pl.ds((3 * k + 0) * CH + off, L)]
                        + 0.587 * buf[pl.ds((3 * k + 1) * CH + off, L)]
                        + 0.114 * buf[pl.ds((3 * k + 2) * CH + off, L)])

            pr = luma(0)
            tr = luma(1)
            l16 = luma(2) * 16.0
            b32 = l16.astype(jnp.int32)
            b32 = jnp.minimum(jnp.maximum(b32, 0), 15)
            valid = (l16 >= 0.0) & (l16 < 16.0)
            idx = b32 * L + lane
            plsc.addupdate_scatter(acc, [idx], ones, mask=valid)
            plsc.addupdate_scatter(acc, [idx + 256], pr, mask=valid)
            plsc.addupdate_scatter(acc, [idx + 512], tr, mask=valid)
            return _

        lax.fori_loop(0, VREGS, px, None)

    pending = start_dmas(0)
    for step in range(STEPS):
        nxt = start_dmas(step + 1) if step + 1 < STEPS else []
        for cp in pending:
            cp.wait()
        compute(step)
        pending = nxt

    pltpu.sync_copy(acc, out_ref.at[wid])


def _finish_body(x_ref, o_ref):
    x = x_ref[...]
    s = jnp.sum(x, axis=(0, 3))
    cnt = s[0:1, :]
    psum = s[1:2, :]
    tsum = s[2:3, :]
    safe = jnp.maximum(cnt, 1.0)
    d = jnp.where(cnt > 0.0, jnp.abs(psum / safe - tsum / safe),
                  jnp.zeros_like(cnt))
    o_ref[...] = jnp.sum(d, axis=1, keepdims=True) / 16.0


def kernel(pred, target, input_img):
    p2 = pred.reshape(NUM_IMGS * 3, SPATIAL)
    t2 = target.reshape(NUM_IMGS * 3, SPATIAL)
    i2 = input_img.reshape(NUM_IMGS * 3, SPATIAL)

    mesh = plsc.VectorSubcoreMesh(core_axis_name="c", subcore_axis_name="s",
                                  num_cores=NC, num_subcores=NS)
    partials = pl.kernel(
        _sc_body,
        out_type=jax.ShapeDtypeStruct((NW, 768), jnp.float32),
        mesh=mesh,
        compiler_params=pltpu.CompilerParams(needs_layout_passes=False),
        scratch_types=[
            pltpu.VMEM((9 * CH,), jnp.float32),
            pltpu.VMEM((9 * CH,), jnp.float32),
            pltpu.VMEM((768,), jnp.float32),
            pltpu.SemaphoreType.DMA,
            pltpu.SemaphoreType.DMA,
        ],
    )(p2, t2, i2)

    loss = pl.pallas_call(
        _finish_body,
        out_shape=jax.ShapeDtypeStruct((1, 1), jnp.float32),
    )(partials.reshape(NW, 3, 16, 16))
    return loss[0, 0]


# speedup vs baseline: 1.1380x; 1.1380x over previous
import functools

import jax
import jax.numpy as jnp
from jax import lax
from jax.experimental import pallas as pl
from jax.experimental.pallas import tpu as pltpu
from jax.experimental.pallas import tpu_sc as plsc

NC = 2
NS = 16
L = 16
NW = NC * NS

NUM_IMGS = 16
SPATIAL = 512 * 512
COLS_PER_TILE = SPATIAL // NW
CH = 4096
HALVES = COLS_PER_TILE // CH
STEPS = NUM_IMGS * HALVES
VREGS = CH // L


def _sc_body(pred_ref, target_ref, input_ref, out_ref, buf0, buf1, acc,
             sem0, sem1):
    wid = lax.axis_index("s") * NC + lax.axis_index("c")
    col_base = pl.multiple_of(wid * COLS_PER_TILE, 8)
    arrs = (pred_ref, target_ref, input_ref)
    bufs = (buf0, buf1)
    sems = (sem0, sem1)

    zero = jnp.zeros((L,), jnp.float32)
    for j in range(768 // L):
        acc[pl.ds(j * L, L)] = zero

    lane = lax.iota(jnp.int32, L)
    ones = jnp.ones((L,), jnp.float32)

    def start_dmas(n, h, slot):
        col0 = pl.multiple_of(col_base + h * CH, 8)
        for k in range(3):
            for c in range(3):
                pltpu.make_async_copy(
                    arrs[k].at[n * 3 + c, pl.ds(col0, CH)],
                    bufs[slot].at[pl.ds((k * 3 + c) * CH, CH)],
                    sems[slot]).start()

    def wait_dmas(slot):
        for j in range(9):
            pltpu.make_async_copy(
                arrs[0].at[0, pl.ds(0, CH)],
                bufs[slot].at[pl.ds(j * CH, CH)],
                sems[slot]).wait()

    U = 8

    def compute(slot):
        buf = bufs[slot]

        def px(j, _):
            base = j * (U * L)
            for u in range(U):
                off = base + u * L

                def ld(row):
                    return buf[pl.ds(row * CH + off, L)]

                pr = 0.299 * ld(0) + 0.587 * ld(1) + 0.114 * ld(2)
                tr = 0.299 * ld(3) + 0.587 * ld(4) + 0.114 * ld(5)
                l16 = 4.784 * ld(6) + 9.392 * ld(7) + 1.824 * ld(8)
                b32 = l16.astype(jnp.int32)
                valid = l16 < 16.0
                idx = b32 * L + lane
                plsc.addupdate_scatter(acc, [idx], ones, mask=valid)
                plsc.addupdate_scatter(acc, [idx + 256], pr, mask=valid)
                plsc.addupdate_scatter(acc, [idx + 512], tr, mask=valid)
            return _

        lax.fori_loop(0, VREGS // U, px, None)

    start_dmas(0, 0, 0)

    def step_body(t, _):
        start_dmas(t, 1, 1)
        wait_dmas(0)
        compute(0)

        @pl.when(t + 1 < NUM_IMGS)
        def _start_next():
            start_dmas(t + 1, 0, 0)

        wait_dmas(1)
        compute(1)
        return _

    lax.fori_loop(0, NUM_IMGS, step_body, None)

    pltpu.sync_copy(acc, out_ref.at[wid])


def _finish_body(x_ref, o_ref):
    x = x_ref[...]
    s = jnp.sum(x, axis=(0, 3))
    cnt = s[0:1, :]
    psum = s[1:2, :]
    tsum = s[2:3, :]
    safe = jnp.maximum(cnt, 1.0)
    d = jnp.where(cnt > 0.0, jnp.abs(psum / safe - tsum / safe),
                  jnp.zeros_like(cnt))
    o_ref[...] = jnp.sum(d, axis=1, keepdims=True) / 16.0


def kernel(pred, target, input_img):
    p2 = pred.reshape(NUM_IMGS * 3, SPATIAL)
    t2 = target.reshape(NUM_IMGS * 3, SPATIAL)
    i2 = input_img.reshape(NUM_IMGS * 3, SPATIAL)

    mesh = plsc.VectorSubcoreMesh(core_axis_name="c", subcore_axis_name="s",
                                  num_cores=NC, num_subcores=NS)
    partials = pl.kernel(
        _sc_body,
        out_type=jax.ShapeDtypeStruct((NW, 768), jnp.float32),
        mesh=mesh,
        compiler_params=pltpu.CompilerParams(needs_layout_passes=False),
        scratch_types=[
            pltpu.VMEM((9 * CH,), jnp.float32),
            pltpu.VMEM((9 * CH,), jnp.float32),
            pltpu.VMEM((768,), jnp.float32),
            pltpu.SemaphoreType.DMA,
            pltpu.SemaphoreType.DMA,
        ],
    )(p2, t2, i2)

    loss = pl.pallas_call(
        _finish_body,
        out_shape=jax.ShapeDtypeStruct((1, 1), jnp.float32),
    )(partials.reshape(NW, 3, 16, 16))
    return loss[0, 0]


# speedup vs baseline: 2.0978x; 1.8434x over previous
import functools

import jax
import jax.numpy as jnp
from jax import lax
from jax.experimental import pallas as pl
from jax.experimental.pallas import tpu as pltpu
from jax.experimental.pallas import tpu_sc as plsc

NC = 2
NS = 16
L = 16
NW = NC * NS

NUM_IMGS = 16
SPATIAL = 512 * 512
NPIX = NUM_IMGS * SPATIAL
PIX_PER_TILE = NPIX // NW
CH = 16384
STEPS = PIX_PER_TILE // CH
U = 8
TBL = 17 * L


def _tc_stage_body(p_ref, t_ref, x_ref, d_ref, b_ref):
    p = p_ref[0]
    t = t_ref[0]
    x = x_ref[0]
    d = (0.299 * (p[0] - t[0]) + 0.587 * (p[1] - t[1])
         + 0.114 * (p[2] - t[2]))
    l16 = 4.784 * x[0] + 9.392 * x[1] + 1.824 * x[2]
    b = jnp.minimum(l16.astype(jnp.int32), 16)
    for ct in range(4):
        sl = slice(128 * ct, 128 * (ct + 1))
        d_ref[0, :, ct] = d[:, sl].reshape(64, 8, 128)
        b_ref[0, :, ct] = b[:, sl].reshape(64, 8, 128)


def _sc_body(d_hbm, b_hbm, out_ref, dbuf0, dbuf1, bbuf0, bbuf1, acc,
             sem0, sem1):
    wid = lax.axis_index("s") * NC + lax.axis_index("c")
    base = pl.multiple_of(wid * PIX_PER_TILE, 8)
    dbufs = (dbuf0, dbuf1)
    bbufs = (bbuf0, bbuf1)
    sems = (sem0, sem1)

    zero = jnp.zeros((L,), jnp.float32)
    for j in range(2 * TBL // L):
        acc[pl.ds(j * L, L)] = zero

    lane = lax.iota(jnp.int32, L)
    ones = jnp.ones((L,), jnp.float32)

    def start_dmas(step, slot):
        off = pl.multiple_of(base + step * CH, 8)
        pltpu.make_async_copy(d_hbm.at[pl.ds(off, CH)], dbufs[slot],
                              sems[slot]).start()
        pltpu.make_async_copy(b_hbm.at[pl.ds(off, CH)], bbufs[slot],
                              sems[slot]).start()

    def wait_dmas(slot):
        pltpu.make_async_copy(d_hbm.at[pl.ds(0, CH)], dbufs[slot],
                              sems[slot]).wait()
        pltpu.make_async_copy(b_hbm.at[pl.ds(0, CH)], bbufs[slot],
                              sems[slot]).wait()

    def compute(slot):
        dbuf = dbufs[slot]
        bbuf = bbufs[slot]

        def px(j, _):
            vbase = j * (U * L)
            for u in range(U):
                off = vbase + u * L
                dv = dbuf[pl.ds(off, L)]
                bv = bbuf[pl.ds(off, L)]
                idx = bv * L + lane
                plsc.addupdate_scatter(acc, [idx], ones)
                plsc.addupdate_scatter(acc, [idx + TBL], dv)
            return _

        lax.fori_loop(0, CH // (U * L), px, None)

    start_dmas(0, 0)

    def step_body(s, _):
        t0 = s * 2
        start_dmas(t0 + 1, 1)
        wait_dmas(0)
        compute(0)

        @pl.when(t0 + 2 < STEPS)
        def _start_next():
            start_dmas(t0 + 2, 0)

        wait_dmas(1)
        compute(1)
        return _

    lax.fori_loop(0, STEPS // 2, step_body, None)

    pltpu.sync_copy(acc, out_ref.at[wid])


def _finish_body(x_ref, o_ref):
    x = x_ref[...]
    s = jnp.sum(x, axis=(0, 3))
    cnt = s[0:1, 0:16]
    dsum = s[1:2, 0:16]
    safe = jnp.maximum(cnt, 1.0)
    d = jnp.where(cnt > 0.0, jnp.abs(dsum) / safe, jnp.zeros_like(cnt))
    o_ref[...] = jnp.sum(d, axis=1, keepdims=True) / 16.0


def kernel(pred, target, input_img):
    f32 = jnp.float32
    diff, bins = pl.pallas_call(
        _tc_stage_body,
        grid=(NUM_IMGS,),
        in_specs=[
            pl.BlockSpec((1, 3, 512, 512), lambda n: (n, 0, 0, 0)),
            pl.BlockSpec((1, 3, 512, 512), lambda n: (n, 0, 0, 0)),
            pl.BlockSpec((1, 3, 512, 512), lambda n: (n, 0, 0, 0)),
        ],
        out_specs=[
            pl.BlockSpec((1, 64, 4, 8, 128), lambda n: (n, 0, 0, 0, 0)),
            pl.BlockSpec((1, 64, 4, 8, 128), lambda n: (n, 0, 0, 0, 0)),
        ],
        out_shape=[
            jax.ShapeDtypeStruct((NUM_IMGS, 64, 4, 8, 128), f32),
            jax.ShapeDtypeStruct((NUM_IMGS, 64, 4, 8, 128), jnp.int32),
        ],
    )(pred, target, input_img)

    mesh = plsc.VectorSubcoreMesh(core_axis_name="c", subcore_axis_name="s",
                                  num_cores=NC, num_subcores=NS)
    partials = pl.kernel(
        _sc_body,
        out_type=jax.ShapeDtypeStruct((NW, 2 * TBL), f32),
        mesh=mesh,
        compiler_params=pltpu.CompilerParams(needs_layout_passes=False),
        scratch_types=[
            pltpu.VMEM((CH,), f32),
            pltpu.VMEM((CH,), f32),
            pltpu.VMEM((CH,), jnp.int32),
            pltpu.VMEM((CH,), jnp.int32),
            pltpu.VMEM((2 * TBL,), f32),
            pltpu.SemaphoreType.DMA,
            pltpu.SemaphoreType.DMA,
        ],
    )(diff.reshape(NPIX), bins.reshape(NPIX))

    loss = pl.pallas_call(
        _finish_body,
        out_shape=jax.ShapeDtypeStruct((1, 1), f32),
    )(partials.reshape(NW, 2, 17, 16))
    return loss[0, 0]


# speedup vs baseline: 2.1123x; 1.0069x over previous
import functools

import jax
import jax.numpy as jnp
from jax import lax
from jax.experimental import pallas as pl
from jax.experimental.pallas import tpu as pltpu
from jax.experimental.pallas import tpu_sc as plsc

NC = 2
NS = 16
L = 16
NW = NC * NS

NUM_IMGS = 16
SPATIAL = 512 * 512
NPIX = NUM_IMGS * SPATIAL
PIX_PER_TILE = NPIX // NW
CH = 16384
STEPS = PIX_PER_TILE // CH
U = 8
TBL = 17 * L


def _tc_stage_body(p_ref, t_ref, x_ref, e_ref):
    p = p_ref[0]
    t = t_ref[0]
    x = x_ref[0]
    d = (0.299 * (p[0] - t[0]) + 0.587 * (p[1] - t[1])
         + 0.114 * (p[2] - t[2]))
    l16 = 4.784 * x[0] + 9.392 * x[1] + 1.824 * x[2]
    b = jnp.minimum(l16.astype(jnp.int32), 16)
    enc = (lax.bitcast_convert_type(d, jnp.int32) & ~31) | b
    for ct in range(4):
        sl = slice(128 * ct, 128 * (ct + 1))
        e_ref[0, :, ct] = enc[:, sl].reshape(64, 8, 128)


def _sc_body(e_hbm, out_ref, ebuf0, ebuf1, acc, sem0, sem1):
    wid = lax.axis_index("s") * NC + lax.axis_index("c")
    base = pl.multiple_of(wid * PIX_PER_TILE, 8)
    ebufs = (ebuf0, ebuf1)
    sems = (sem0, sem1)

    zero = jnp.zeros((L,), jnp.float32)
    for j in range(4 * TBL // L):
        acc[pl.ds(j * L, L)] = zero

    lane = lax.iota(jnp.int32, L)
    ones = jnp.ones((L,), jnp.float32)

    def start_dmas(step, slot):
        off = pl.multiple_of(base + step * CH, 8)
        pltpu.make_async_copy(e_hbm.at[pl.ds(off, CH)], ebufs[slot],
                              sems[slot]).start()

    def wait_dmas(slot):
        pltpu.make_async_copy(e_hbm.at[pl.ds(0, CH)], ebufs[slot],
                              sems[slot]).wait()

    def compute(slot):
        ebuf = ebufs[slot]

        def px(j, _):
            vbase = j * (U * L)
            for u in range(U):
                off = vbase + u * L
                ev = ebuf[pl.ds(off, L)]
                idx = (ev & 31) * L + lane
                dv = plsc.bitcast(ev & ~31, jnp.float32)
                tb = 2 * TBL * (u % 2)
                plsc.addupdate_scatter(acc, [idx + tb], ones)
                plsc.addupdate_scatter(acc, [idx + (tb + TBL)], dv)
            return _

        lax.fori_loop(0, CH // (U * L), px, None)

    start_dmas(0, 0)

    def step_body(s, _):
        t0 = s * 2
        start_dmas(t0 + 1, 1)
        wait_dmas(0)
        compute(0)

        @pl.when(t0 + 2 < STEPS)
        def _start_next():
            start_dmas(t0 + 2, 0)

        wait_dmas(1)
        compute(1)
        return _

    lax.fori_loop(0, STEPS // 2, step_body, None)

    pltpu.sync_copy(acc, out_ref.at[wid])


def _finish_body(x_ref, o_ref):
    x = x_ref[...]
    s = jnp.sum(x, axis=(0, 1, 4))
    cnt = s[0:1, 0:16]
    dsum = s[1:2, 0:16]
    safe = jnp.maximum(cnt, 1.0)
    d = jnp.where(cnt > 0.0, jnp.abs(dsum) / safe, jnp.zeros_like(cnt))
    o_ref[...] = jnp.sum(d, axis=1, keepdims=True) / 16.0


def kernel(pred, target, input_img):
    f32 = jnp.float32
    enc = pl.pallas_call(
        _tc_stage_body,
        grid=(NUM_IMGS,),
        in_specs=[
            pl.BlockSpec((1, 3, 512, 512), lambda n: (n, 0, 0, 0)),
            pl.BlockSpec((1, 3, 512, 512), lambda n: (n, 0, 0, 0)),
            pl.BlockSpec((1, 3, 512, 512), lambda n: (n, 0, 0, 0)),
        ],
        out_specs=pl.BlockSpec((1, 64, 4, 8, 128), lambda n: (n, 0, 0, 0, 0)),
        out_shape=jax.ShapeDtypeStruct((NUM_IMGS, 64, 4, 8, 128), jnp.int32),
    )(pred, target, input_img)

    mesh = plsc.VectorSubcoreMesh(core_axis_name="c", subcore_axis_name="s",
                                  num_cores=NC, num_subcores=NS)
    partials = pl.kernel(
        _sc_body,
        out_type=jax.ShapeDtypeStruct((NW, 4 * TBL), f32),
        mesh=mesh,
        compiler_params=pltpu.CompilerParams(needs_layout_passes=False),
        scratch_types=[
            pltpu.VMEM((CH,), jnp.int32),
            pltpu.VMEM((CH,), jnp.int32),
            pltpu.VMEM((4 * TBL,), f32),
            pltpu.SemaphoreType.DMA,
            pltpu.SemaphoreType.DMA,
        ],
    )(enc.reshape(NPIX))

    loss = pl.pallas_call(
        _finish_body,
        out_shape=jax.ShapeDtypeStruct((1, 1), f32),
    )(partials.reshape(NW, 2, 2, 17, 16))
    return loss[0, 0]


# speedup vs baseline: 2.1124x; 1.0000x over previous
import functools

import jax
import jax.numpy as jnp
from jax import lax
from jax.experimental import pallas as pl
from jax.experimental.pallas import tpu as pltpu
from jax.experimental.pallas import tpu_sc as plsc

NC = 2
NS = 16
L = 16
NW = NC * NS

NUM_IMGS = 16
SPATIAL = 512 * 512
NPIX = NUM_IMGS * SPATIAL
PIX_PER_TILE = NPIX // NW
CH = 16384
STEPS = PIX_PER_TILE // CH
U = 8
TBL = 17 * L


def _tc_stage_body(p_ref, t_ref, x_ref, e_ref):
    p = p_ref[0]
    t = t_ref[0]
    x = x_ref[0]
    d = (0.299 * (p[0] - t[0]) + 0.587 * (p[1] - t[1])
         + 0.114 * (p[2] - t[2]))
    l16 = 4.784 * x[0] + 9.392 * x[1] + 1.824 * x[2]
    b = jnp.minimum(l16.astype(jnp.int32), 16)
    enc = (lax.bitcast_convert_type(d, jnp.int32) & ~31) | b
    for ct in range(4):
        sl = slice(128 * ct, 128 * (ct + 1))
        e_ref[0, :, ct] = enc[:, sl].reshape(64, 8, 128)


def _sc_body(e_hbm, out_ref, ebuf0, ebuf1, acc_c0, acc_d0, acc_c1, acc_d1,
             stage, sem0, sem1):
    wid = lax.axis_index("s") * NC + lax.axis_index("c")
    base = pl.multiple_of(wid * PIX_PER_TILE, 8)
    ebufs = (ebuf0, ebuf1)
    sems = (sem0, sem1)
    accs = (acc_c0, acc_d0, acc_c1, acc_d1)

    zero = jnp.zeros((L,), jnp.float32)
    for acc in accs:
        for j in range(TBL // L):
            acc[pl.ds(j * L, L)] = zero

    lane = lax.iota(jnp.int32, L)
    ones = jnp.ones((L,), jnp.float32)

    def start_dmas(step, slot):
        off = pl.multiple_of(base + step * CH, 8)
        pltpu.make_async_copy(e_hbm.at[pl.ds(off, CH)], ebufs[slot],
                              sems[slot]).start()

    def wait_dmas(slot):
        pltpu.make_async_copy(e_hbm.at[pl.ds(0, CH)], ebufs[slot],
                              sems[slot]).wait()

    def compute(slot):
        ebuf = ebufs[slot]

        def px(j, _):
            vbase = j * (U * L)
            for u in range(U):
                off = vbase + u * L
                ev = ebuf[pl.ds(off, L)]
                idx = (ev & 31) * L + lane
                dv = plsc.bitcast(ev & ~31, jnp.float32)
                plsc.addupdate_scatter(accs[2 * (u % 2)], [idx], ones)
                plsc.addupdate_scatter(accs[2 * (u % 2) + 1], [idx], dv)
            return _

        lax.fori_loop(0, CH // (U * L), px, None)

    start_dmas(0, 0)

    def step_body(s, _):
        t0 = s * 2
        start_dmas(t0 + 1, 1)
        wait_dmas(0)
        compute(0)

        @pl.when(t0 + 2 < STEPS)
        def _start_next():
            start_dmas(t0 + 2, 0)

        wait_dmas(1)
        compute(1)
        return _

    lax.fori_loop(0, STEPS // 2, step_body, None)

    for q, acc in enumerate(accs):
        for j in range(TBL // L):
            stage[pl.ds(q * TBL + j * L, L)] = acc[pl.ds(j * L, L)]
    pltpu.sync_copy(stage, out_ref.at[wid])


def _finish_body(x_ref, o_ref):
    x = x_ref[...]
    s = jnp.sum(x, axis=(0, 1, 4))
    cnt = s[0:1, 0:16]
    dsum = s[1:2, 0:16]
    safe = jnp.maximum(cnt, 1.0)
    d = jnp.where(cnt > 0.0, jnp.abs(dsum) / safe, jnp.zeros_like(cnt))
    o_ref[...] = jnp.sum(d, axis=1, keepdims=True) / 16.0


def kernel(pred, target, input_img):
    f32 = jnp.float32
    enc = pl.pallas_call(
        _tc_stage_body,
        grid=(NUM_IMGS,),
        in_specs=[
            pl.BlockSpec((1, 3, 512, 512), lambda n: (n, 0, 0, 0)),
            pl.BlockSpec((1, 3, 512, 512), lambda n: (n, 0, 0, 0)),
            pl.BlockSpec((1, 3, 512, 512), lambda n: (n, 0, 0, 0)),
        ],
        out_specs=pl.BlockSpec((1, 64, 4, 8, 128), lambda n: (n, 0, 0, 0, 0)),
        out_shape=jax.ShapeDtypeStruct((NUM_IMGS, 64, 4, 8, 128), jnp.int32),
    )(pred, target, input_img)

    mesh = plsc.VectorSubcoreMesh(core_axis_name="c", subcore_axis_name="s",
                                  num_cores=NC, num_subcores=NS)
    partials = pl.kernel(
        _sc_body,
        out_type=jax.ShapeDtypeStruct((NW, 4 * TBL), f32),
        mesh=mesh,
        compiler_params=pltpu.CompilerParams(needs_layout_passes=False),
        scratch_types=[
            pltpu.VMEM((CH,), jnp.int32),
            pltpu.VMEM((CH,), jnp.int32),
            pltpu.VMEM((TBL,), f32),
            pltpu.VMEM((TBL,), f32),
            pltpu.VMEM((TBL,), f32),
            pltpu.VMEM((TBL,), f32),
            pltpu.VMEM((4 * TBL,), f32),
            pltpu.SemaphoreType.DMA,
            pltpu.SemaphoreType.DMA,
        ],
    )(enc.reshape(NPIX))

    loss = pl.pallas_call(
        _finish_body,
        out_shape=jax.ShapeDtypeStruct((1, 1), f32),
    )(partials.reshape(NW, 2, 2, 17, 16))
    return loss[0, 0]
